# baseline (speedup 1.0000x reference)
import functools

import jax
import jax.numpy as jnp
from jax import lax
from jax.experimental import pallas as pl
from jax.experimental.pallas import tpu as pltpu
from jax.experimental.pallas import tpu_sc as plsc

HIDDEN = 768
EPS = 1e-12
SEQ = 512
CHUNK = 8
NSL = HIDDEN // 16
NTBL = 7
NBUF = NTBL + 1
INV_H = 1.0 / HIDDEN
RSQRT_SEED = jnp.int32(0x5F3759DF)


def _newton_rsqrt(x):
    i = lax.bitcast_convert_type(x, jnp.int32)
    i = RSQRT_SEED - lax.shift_right_arithmetic(i, 1)
    y = lax.bitcast_convert_type(i, jnp.float32)
    for _ in range(3):
        y = y * (1.5 - 0.5 * x * y * y)
    return y


def _sc_fused(ids, x0, y1, x2, y3, word_emb, x_emb, y_emb, h_emb, w_emb,
              pos_tt, ln_w, ln_b):
    n_tok = ids.shape[0]
    info = plsc.get_sparse_core_info()
    n_workers = info.num_cores * info.num_subcores
    per_w = n_tok // n_workers
    n_steps = per_w // CHUNK
    n_half = n_steps // 2

    mesh = plsc.VectorSubcoreMesh(core_axis_name="c", subcore_axis_name="s")

    @functools.partial(
        pl.kernel,
        mesh=mesh,
        out_type=jax.ShapeDtypeStruct((n_tok, HIDDEN), jnp.float32),
        scratch_types=[
            pltpu.VMEM((per_w,), jnp.int32),
            pltpu.VMEM((per_w,), jnp.int32),
            pltpu.VMEM((per_w,), jnp.int32),
            pltpu.VMEM((per_w,), jnp.int32),
            pltpu.VMEM((per_w,), jnp.int32),
            pltpu.VMEM((per_w,), jnp.int32),
            pltpu.VMEM((per_w,), jnp.int32),
            pltpu.VMEM((2, NBUF, CHUNK, HIDDEN), jnp.float32),
            pltpu.VMEM((2, CHUNK, HIDDEN), jnp.float32),
            pltpu.VMEM((HIDDEN,), jnp.float32),
            pltpu.VMEM((HIDDEN,), jnp.float32),
            pltpu.VMEM((2, CHUNK, 16), jnp.float32),
            pltpu.SemaphoreType.DMA,
            pltpu.SemaphoreType.DMA,
            pltpu.SemaphoreType.DMA,
            pltpu.SemaphoreType.DMA,
        ],
    )
    def fused(ids_h, x0_h, y1_h, x2_h, y3_h, word_h, x_h, y_h, h_h, w_h,
              pos_h, lnw_h, lnb_h, out_h, ids_v, x0_v, y1_v, x2_v, y3_v,
              hx_v, wx_v, gset, obuf, wbuf, bbuf, stat, gsem0, gsem1,
              wsem0, wsem1):
        wid = lax.axis_index("s") * info.num_cores + lax.axis_index("c")
        base = wid * per_w

        pltpu.sync_copy(ids_h.at[pl.ds(base, per_w)], ids_v)
        pltpu.sync_copy(x0_h.at[pl.ds(base, per_w)], x0_v)
        pltpu.sync_copy(y1_h.at[pl.ds(base, per_w)], y1_v)
        pltpu.sync_copy(x2_h.at[pl.ds(base, per_w)], x2_v)
        pltpu.sync_copy(y3_h.at[pl.ds(base, per_w)], y3_v)
        pltpu.sync_copy(lnw_h, wbuf)
        pltpu.sync_copy(lnb_h, bbuf)

        def hw_body(j, _):
            s = pl.ds(j * 16, 16)
            hx_v[s] = y3_v[s] - y1_v[s]
            wx_v[s] = x2_v[s] - x0_v[s]
            return 0

        lax.fori_loop(0, per_w // 16, hw_body, 0)

        tables = (
            (word_h, ids_v),
            (x_h, x0_v),
            (x_h, x2_v),
            (y_h, y1_v),
            (y_h, y3_v),
            (h_h, hx_v),
            (w_h, wx_v),
        )
        gsems = (gsem0, gsem1)
        wsems = (wsem0, wsem1)

        def fire_gathers(q, s):
            sl = pl.ds(pl.multiple_of(s * CHUNK, CHUNK), CHUNK)
            for k in range(NTBL):
                tbl, iv = tables[k]
                pltpu.async_copy(tbl.at[iv.at[sl]], gset.at[q, k], gsems[q])
            poff = pl.multiple_of(jnp.bitwise_and(s * CHUNK, SEQ - 1), CHUNK)
            pltpu.async_copy(pos_h.at[pl.ds(poff, CHUNK)], gset.at[q, NTBL],
                             gsems[q])

        def drain_gathers(q):
            dummy = word_h.at[pl.ds(0, CHUNK)]
            for k in range(NBUF):
                pltpu.make_async_copy(dummy, gset.at[q, k], gsems[q]).wait()

        def wait_wb(q):
            pltpu.make_async_copy(obuf.at[q], out_h.at[pl.ds(0, CHUNK)],
                                  wsems[q]).wait()

        lanes = lax.iota(jnp.int32, 16)

        def lane_sum(v):
            for k in (8, 4, 2, 1):
                idx = jnp.bitwise_xor(lanes, k)
                v = v + v.at[idx].get(mode="promise_in_bounds")
            return v

        def accum(q):
            def rbody(r, _):
                acc_s0 = jnp.zeros((16,), jnp.float32)
                acc_s1 = jnp.zeros((16,), jnp.float32)
                acc_q0 = jnp.zeros((16,), jnp.float32)
                acc_q1 = jnp.zeros((16,), jnp.float32)
                for c in range(NSL):
                    cs = pl.ds(c * 16, 16)
                    g = [gset[q, k, r, cs] for k in range(NBUF)]
                    v = (((g[0] + g[1]) + (g[2] + g[3]))
                         + ((g[4] + g[5]) + (g[6] + g[7])))
                    obuf[q, r, cs] = v
                    if c % 2 == 0:
                        acc_s0 = acc_s0 + v
                        acc_q0 = acc_q0 + v * v
                    else:
                        acc_s1 = acc_s1 + v
                        acc_q1 = acc_q1 + v * v
                mean = lane_sum(acc_s0 + acc_s1) * INV_H
                var = lane_sum(acc_q0 + acc_q1) * INV_H - mean * mean
                rstd = _newton_rsqrt(var + EPS)
                stat[0, r] = rstd
                stat[1, r] = mean * rstd
                return 0

            lax.fori_loop(0, CHUNK, rbody, 0)

            svals = [stat[0, r] for r in range(CHUNK)]
            tvals = [stat[1, r] for r in range(CHUNK)]

            def cbody(c, _):
                cs = pl.ds(c * 16, 16)
                wv = wbuf[cs]
                bv = bbuf[cs]
                for r in range(CHUNK):
                    v = obuf[q, r, cs]
                    obuf[q, r, cs] = (v * svals[r] - tvals[r]) * wv + bv
                return 0

            lax.fori_loop(0, NSL, cbody, 0)

        def fire_wb(q, s):
            dst0 = pl.multiple_of(base + s * CHUNK, CHUNK)
            pltpu.async_copy(obuf.at[q], out_h.at[pl.ds(dst0, CHUNK)],
                             wsems[q])

        fire_gathers(0, 0)

        def gbody(g, _):
            s0 = 2 * g
            drain_gathers(0)
            fire_gathers(1, s0 + 1)
            pl.when(g > 0)(lambda: wait_wb(0))
            accum(0)
            fire_wb(0, s0)

            drain_gathers(1)
            pl.when(g < n_half - 1)(lambda: fire_gathers(0, s0 + 2))
            pl.when(g > 0)(lambda: wait_wb(1))
            accum(1)
            fire_wb(1, s0 + 1)
            return 0

        lax.fori_loop(0, n_half, gbody, 0)
        wait_wb(0)
        wait_wb(1)

    return fused(ids, x0, y1, x2, y3, word_emb, x_emb, y_emb, h_emb, w_emb,
                 pos_tt, ln_w, ln_b)


def kernel(input_ids, bbox, word_emb, pos_emb, x_emb, y_emb, h_emb, w_emb,
           tt_emb, ln_w, ln_b):
    batch, seq = input_ids.shape
    n_tok = batch * seq
    ids = input_ids.reshape(n_tok).astype(jnp.int32)
    bb = bbox.reshape(n_tok, 4).astype(jnp.int32)
    x0, y1, x2, y3 = bb[:, 0], bb[:, 1], bb[:, 2], bb[:, 3]
    pos_tt = pos_emb + tt_emb[0][None, :]
    out = _sc_fused(ids, x0, y1, x2, y3, word_emb, x_emb, y_emb, h_emb,
                    w_emb, pos_tt, ln_w, ln_b)
    return out.reshape(batch, seq, HIDDEN)


# speedup vs baseline: 1.2711x; 1.2711x over previous
import functools

import jax
import jax.numpy as jnp
from jax import lax
from jax.experimental import pallas as pl
from jax.experimental.pallas import tpu as pltpu
from jax.experimental.pallas import tpu_sc as plsc

HIDDEN = 768
EPS = 1e-12
SEQ = 512
CHUNK = 8
NSL = HIDDEN // 16
NTBL = 7
NBUF = NTBL + 1
INV_H = 1.0 / HIDDEN
RSQRT_SEED = jnp.int32(0x5F3759DF)


def _newton_rsqrt(x):
    i = lax.bitcast_convert_type(x, jnp.int32)
    i = RSQRT_SEED - lax.shift_right_arithmetic(i, 1)
    y = lax.bitcast_convert_type(i, jnp.float32)
    for _ in range(3):
        y = y * (1.5 - 0.5 * x * y * y)
    return y


def _sc_fused(ids, x0, y1, x2, y3, word_emb, x_emb, y_emb, h_emb, w_emb,
              pos_tt, ln_w, ln_b):
    n_tok = ids.shape[0]
    info = plsc.get_sparse_core_info()
    n_workers = info.num_cores * info.num_subcores
    per_w = n_tok // n_workers
    n_steps = per_w // CHUNK
    n_half = n_steps // 2

    mesh = plsc.VectorSubcoreMesh(core_axis_name="c", subcore_axis_name="s")

    @functools.partial(
        pl.kernel,
        mesh=mesh,
        out_type=jax.ShapeDtypeStruct((n_tok, HIDDEN), jnp.float32),
        scratch_types=[
            pltpu.VMEM((per_w,), jnp.int32),
            pltpu.VMEM((per_w,), jnp.int32),
            pltpu.VMEM((per_w,), jnp.int32),
            pltpu.VMEM((per_w,), jnp.int32),
            pltpu.VMEM((per_w,), jnp.int32),
            pltpu.VMEM((per_w,), jnp.int32),
            pltpu.VMEM((per_w,), jnp.int32),
            pltpu.VMEM((2, NBUF, CHUNK, HIDDEN), jnp.float32),
            pltpu.VMEM((2, CHUNK, HIDDEN), jnp.float32),
            pltpu.VMEM((HIDDEN,), jnp.float32),
            pltpu.VMEM((HIDDEN,), jnp.float32),
            pltpu.VMEM((2, CHUNK, 16), jnp.float32),
            pltpu.SemaphoreType.DMA,
            pltpu.SemaphoreType.DMA,
            pltpu.SemaphoreType.DMA,
            pltpu.SemaphoreType.DMA,
        ],
    )
    def fused(ids_h, x0_h, y1_h, x2_h, y3_h, word_h, x_h, y_h, h_h, w_h,
              pos_h, lnw_h, lnb_h, out_h, ids_v, x0_v, y1_v, x2_v, y3_v,
              hx_v, wx_v, gset, obuf, wbuf, bbuf, stat, gsem0, gsem1,
              wsem0, wsem1):
        wid = lax.axis_index("s") * info.num_cores + lax.axis_index("c")
        base = wid * per_w

        pltpu.sync_copy(ids_h.at[pl.ds(base, per_w)], ids_v)
        pltpu.sync_copy(x0_h.at[pl.ds(base, per_w)], x0_v)
        pltpu.sync_copy(y1_h.at[pl.ds(base, per_w)], y1_v)
        pltpu.sync_copy(x2_h.at[pl.ds(base, per_w)], x2_v)
        pltpu.sync_copy(y3_h.at[pl.ds(base, per_w)], y3_v)
        pltpu.sync_copy(lnw_h, wbuf)
        pltpu.sync_copy(lnb_h, bbuf)

        def hw_body(j, _):
            s = pl.ds(j * 16, 16)
            hx_v[s] = y3_v[s] - y1_v[s]
            wx_v[s] = x2_v[s] - x0_v[s]
            return 0

        lax.fori_loop(0, per_w // 16, hw_body, 0)

        tables = (
            (word_h, ids_v),
            (x_h, x0_v),
            (x_h, x2_v),
            (y_h, y1_v),
            (y_h, y3_v),
            (h_h, hx_v),
            (w_h, wx_v),
        )
        gsems = (gsem0, gsem1)
        wsems = (wsem0, wsem1)

        def fire_gathers(q, s):
            sl = pl.ds(pl.multiple_of(s * CHUNK, CHUNK), CHUNK)
            for k in range(NTBL):
                tbl, iv = tables[k]
                pltpu.async_copy(tbl.at[iv.at[sl]], gset.at[q, k], gsems[q])
            poff = pl.multiple_of(jnp.bitwise_and(s * CHUNK, SEQ - 1), CHUNK)
            pltpu.async_copy(pos_h.at[pl.ds(poff, CHUNK)], gset.at[q, NTBL],
                             gsems[q])

        def drain_gathers(q):
            dummy = word_h.at[pl.ds(0, CHUNK)]
            for k in range(NBUF):
                pltpu.make_async_copy(dummy, gset.at[q, k], gsems[q]).wait()

        def wait_wb(q):
            pltpu.make_async_copy(obuf.at[q], out_h.at[pl.ds(0, CHUNK)],
                                  wsems[q]).wait()

        lanes = lax.iota(jnp.int32, 16)

        def lane_sum(v):
            for k in (8, 4, 2, 1):
                idx = jnp.bitwise_xor(lanes, k)
                v = v + v.at[idx].get(mode="promise_in_bounds")
            return v

        def accum(q):
            def rbody(r, _):
                acc_s = jnp.zeros((16,), jnp.float32)
                acc_q = jnp.zeros((16,), jnp.float32)
                for c in range(NSL):
                    cs = pl.ds(c * 16, 16)
                    v = gset[q, 0, r, cs]
                    for k in range(1, NBUF):
                        v = v + gset[q, k, r, cs]
                    obuf[q, r, cs] = v
                    acc_s = acc_s + v
                    acc_q = acc_q + v * v
                mean = lane_sum(acc_s) * INV_H
                var = lane_sum(acc_q) * INV_H - mean * mean
                rstd = _newton_rsqrt(var + EPS)
                stat[0, r] = rstd
                stat[1, r] = mean * rstd
                return 0

            lax.fori_loop(0, CHUNK, rbody, 0)

            svals = [stat[0, r] for r in range(CHUNK)]
            tvals = [stat[1, r] for r in range(CHUNK)]

            def cbody(c, _):
                cs = pl.ds(c * 16, 16)
                wv = wbuf[cs]
                bv = bbuf[cs]
                for r in range(CHUNK):
                    v = obuf[q, r, cs]
                    obuf[q, r, cs] = (v * svals[r] - tvals[r]) * wv + bv
                return 0

            lax.fori_loop(0, NSL, cbody, 0)

        def fire_wb(q, s):
            dst0 = pl.multiple_of(base + s * CHUNK, CHUNK)
            pltpu.async_copy(obuf.at[q], out_h.at[pl.ds(dst0, CHUNK)],
                             wsems[q])

        fire_gathers(0, 0)

        def gbody(g, _):
            s0 = 2 * g
            drain_gathers(0)
            fire_gathers(1, s0 + 1)
            pl.when(g > 0)(lambda: wait_wb(0))
            accum(0)
            fire_wb(0, s0)

            drain_gathers(1)
            pl.when(g < n_half - 1)(lambda: fire_gathers(0, s0 + 2))
            pl.when(g > 0)(lambda: wait_wb(1))
            accum(1)
            fire_wb(1, s0 + 1)
            return 0

        lax.fori_loop(0, n_half, gbody, 0)
        wait_wb(0)
        wait_wb(1)

    return fused(ids, x0, y1, x2, y3, word_emb, x_emb, y_emb, h_emb, w_emb,
                 pos_tt, ln_w, ln_b)


def kernel(input_ids, bbox, word_emb, pos_emb, x_emb, y_emb, h_emb, w_emb,
           tt_emb, ln_w, ln_b):
    batch, seq = input_ids.shape
    n_tok = batch * seq
    ids = input_ids.reshape(n_tok).astype(jnp.int32)
    bb = bbox.reshape(n_tok, 4).astype(jnp.int32)
    x0, y1, x2, y3 = bb[:, 0], bb[:, 1], bb[:, 2], bb[:, 3]
    pos_tt = pos_emb + tt_emb[0][None, :]
    out = _sc_fused(ids, x0, y1, x2, y3, word_emb, x_emb, y_emb, h_emb,
                    w_emb, pos_tt, ln_w, ln_b)
    return out.reshape(batch, seq, HIDDEN)
